# baseline (speedup 1.0000x reference)
import functools
import jax
import jax.numpy as jnp
from jax import lax
from jax.experimental import pallas as pl
from jax.experimental.pallas import tpu as pltpu
from jax.experimental.pallas import tpu_sc as plsc

K = 100000
M = 64
NC = 2
NS = 16
NW = NC * NS
L = 16
CHT = 2048
SUP = 10


def _make_lookup(J, I):
    n_chunks = (J * I) // CHT
    cpj = I // CHT
    assert n_chunks % SUP == 0 and I % CHT == 0
    mesh = plsc.VectorSubcoreMesh(core_axis_name="c", subcore_axis_name="s")

    @functools.partial(
        pl.kernel,
        out_type=jax.ShapeDtypeStruct((J, M, I), jnp.float32),
        mesh=mesh,
        compiler_params=pltpu.CompilerParams(
            use_tc_tiling_on_sc=True, needs_layout_passes=False
        ),
        scratch_types=[
            pltpu.VMEM((K,), jnp.int32),
            pltpu.VMEM((CHT,), jnp.int32),
            pltpu.VMEM((CHT,), jnp.int32),
            pltpu.VMEM((CHT,), jnp.float32),
            pltpu.VMEM((CHT,), jnp.float32),
            pltpu.VMEM((CHT,), jnp.float32),
            pltpu.VMEM((CHT,), jnp.float32),
            pltpu.SemaphoreType.DMA,
            pltpu.SemaphoreType.DMA,
            pltpu.SemaphoreType.DMA,
            pltpu.SemaphoreType.DMA,
        ],
    )
    def lookup(yflat_hbm, packed_hbm, out_hbm, row_v, ix0, ix1,
               oa0, oa1, ob0, ob1, is0, is1, os0, os1):
        wid = lax.axis_index("s") * NC + lax.axis_index("c")
        ixb = (ix0, ix1)
        obuf0 = (oa0, oa1)
        obuf1 = (ob0, ob1)
        isem = (is0, is1)
        osem = (os0, os1)
        m0 = wid
        himask = jnp.int32(-65536)

        pltpu.sync_copy(packed_hbm.at[wid], row_v)

        def idx_load(c, b):
            cp = pltpu.make_async_copy(
                yflat_hbm.at[pl.ds(c * CHT, CHT)], ixb[b], isem[b]
            )
            cp.start()
            return cp

        def gather_chunk(b):
            @plsc.parallel_loop(0, CHT // L, unroll=16)
            def _(q):
                iv = ixb[b][pl.ds(q * L, L)]
                g = plsc.load_gather(row_v, [iv])
                lo = lax.shift_left(g, 16)
                hi = lax.bitwise_and(g, himask)
                obuf0[b][pl.ds(q * L, L)] = plsc.bitcast(lo, jnp.float32)
                obuf1[b][pl.ds(q * L, L)] = plsc.bitcast(hi, jnp.float32)

        def out_copies(c, b):
            j = c // cpj
            i0 = (c % cpj) * CHT
            return (
                pltpu.make_async_copy(
                    obuf0[b], out_hbm.at[j, m0, pl.ds(i0, CHT)], osem[b]
                ),
                pltpu.make_async_copy(
                    obuf1[b], out_hbm.at[j, m0 + NW, pl.ds(i0, CHT)], osem[b]
                ),
            )

        def super_body(c0):
            icp = [None] * SUP
            ocp = [None] * SUP
            icp[0] = idx_load(c0, 0)
            for k in range(SUP):
                b = k % 2
                if k + 1 < SUP:
                    icp[k + 1] = idx_load(c0 + k + 1, 1 - b)
                icp[k].wait()
                gather_chunk(b)
                ocp[k] = out_copies(c0 + k, b)
                ocp[k][0].start()
                ocp[k][1].start()
                if k >= 1:
                    ocp[k - 1][0].wait()
                    ocp[k - 1][1].wait()
            ocp[SUP - 1][0].wait()
            ocp[SUP - 1][1].wait()

        pl.loop(0, n_chunks, step=SUP)(super_body)

    return lookup


def kernel(y, table):
    I, J = y.shape
    y_flat = y.T.reshape(I * J).astype(jnp.int32)
    tt_i = lax.bitcast_convert_type(table, jnp.int32).T
    half = jnp.int32(0x8000)
    m0b = tt_i[:NW] + half
    m1b = tt_i[NW:] + half
    packed = jnp.bitwise_or(
        lax.shift_right_logical(m0b, 16),
        jnp.bitwise_and(m1b, jnp.int32(-65536)),
    )
    out_t = _make_lookup(J, I)(y_flat, packed)
    return out_t.transpose(2, 0, 1)


# speedup vs baseline: 1.0054x; 1.0054x over previous
import functools
import jax
import jax.numpy as jnp
from jax import lax
from jax.experimental import pallas as pl
from jax.experimental.pallas import tpu as pltpu
from jax.experimental.pallas import tpu_sc as plsc

K = 100000
M = 64
NC = 2
NS = 16
NW = NC * NS
L = 16
CHT = 2048
SUP = 10


def _make_lookup(J, I):
    n_chunks = (J * I) // CHT
    cpj = I // CHT
    assert n_chunks % SUP == 0 and I % CHT == 0
    mesh = plsc.VectorSubcoreMesh(core_axis_name="c", subcore_axis_name="s")

    @functools.partial(
        pl.kernel,
        out_type=jax.ShapeDtypeStruct((J, M, I), jnp.float32),
        mesh=mesh,
        compiler_params=pltpu.CompilerParams(
            use_tc_tiling_on_sc=True, needs_layout_passes=False
        ),
        scratch_types=[
            pltpu.VMEM((K,), jnp.int32),
            pltpu.VMEM((CHT,), jnp.int32),
            pltpu.VMEM((CHT,), jnp.int32),
            pltpu.VMEM((CHT,), jnp.float32),
            pltpu.VMEM((CHT,), jnp.float32),
            pltpu.VMEM((CHT,), jnp.float32),
            pltpu.VMEM((CHT,), jnp.float32),
            pltpu.SemaphoreType.DMA,
            pltpu.SemaphoreType.DMA,
            pltpu.SemaphoreType.DMA,
            pltpu.SemaphoreType.DMA,
        ],
    )
    def lookup(yflat_hbm, packed_hbm, out_hbm, row_v, ix0, ix1,
               oa0, oa1, ob0, ob1, is0, is1, os0, os1):
        wid = lax.axis_index("s") * NC + lax.axis_index("c")
        ixb = (ix0, ix1)
        obuf0 = (oa0, oa1)
        obuf1 = (ob0, ob1)
        isem = (is0, is1)
        osem = (os0, os1)
        m0 = wid
        himask = jnp.int32(-65536)

        pltpu.sync_copy(packed_hbm.at[wid], row_v)

        def idx_load(c, b):
            cp = pltpu.make_async_copy(
                yflat_hbm.at[pl.ds(c * CHT, CHT)], ixb[b], isem[b]
            )
            cp.start()
            return cp

        def gather_chunk(b):
            @plsc.parallel_loop(0, CHT // L, unroll=8)
            def _(q):
                iv = ixb[b][pl.ds(q * L, L)]
                g = plsc.load_gather(row_v, [iv])
                lo = lax.shift_left(g, 16)
                hi = lax.bitwise_and(g, himask)
                obuf0[b][pl.ds(q * L, L)] = plsc.bitcast(lo, jnp.float32)
                obuf1[b][pl.ds(q * L, L)] = plsc.bitcast(hi, jnp.float32)

        def out_copies(c, b):
            j = c // cpj
            i0 = (c % cpj) * CHT
            return (
                pltpu.make_async_copy(
                    obuf0[b], out_hbm.at[j, m0, pl.ds(i0, CHT)], osem[b]
                ),
                pltpu.make_async_copy(
                    obuf1[b], out_hbm.at[j, m0 + NW, pl.ds(i0, CHT)], osem[b]
                ),
            )

        def super_body(c0):
            icp = [None] * SUP
            ocp = [None] * SUP
            icp[0] = idx_load(c0, 0)
            for k in range(SUP):
                b = k % 2
                if k + 1 < SUP:
                    icp[k + 1] = idx_load(c0 + k + 1, 1 - b)
                icp[k].wait()
                gather_chunk(b)
                ocp[k] = out_copies(c0 + k, b)
                ocp[k][0].start()
                ocp[k][1].start()
                if k >= 1:
                    ocp[k - 1][0].wait()
                    ocp[k - 1][1].wait()
            ocp[SUP - 1][0].wait()
            ocp[SUP - 1][1].wait()

        pl.loop(0, n_chunks, step=SUP)(super_body)

    return lookup


def kernel(y, table):
    I, J = y.shape
    y_flat = y.T.reshape(I * J).astype(jnp.int32)
    tt_i = lax.bitcast_convert_type(table, jnp.int32).T
    half = jnp.int32(0x8000)
    m0b = tt_i[:NW] + half
    m1b = tt_i[NW:] + half
    packed = jnp.bitwise_or(
        lax.shift_right_logical(m0b, 16),
        jnp.bitwise_and(m1b, jnp.int32(-65536)),
    )
    out_t = _make_lookup(J, I)(y_flat, packed)
    return out_t.transpose(2, 0, 1)
